# Initial kernel scaffold; baseline (speedup 1.0000x reference)
import math

import jax
import jax.numpy as jnp
from jax import lax
from jax.experimental import pallas as pl
from jax.experimental.pallas import tpu as pltpu
from jax.experimental.pallas import tpu_sc as plsc

N = 10000
E = 320000
F = 128
H = 4
D = 32

NC = 2
NS = 16
NW = NC * NS
EW = E // NW
CHUNK = 80
NCHUNK = EW // CHUNK
GROUPS = CHUNK // 16
ROWW = 144
RPT = N // NS
ZROWS = 125
INV_SQRT_HD = 1.0 / math.sqrt(H * D)


def _proj_body(x_ref, wq_ref, wkv_ref, q_ref, kv_ref):
    x = x_ref[...]
    q_ref[...] = jnp.dot(x, wq_ref[...], preferred_element_type=jnp.float32)
    kv_ref[...] = jnp.dot(x, wkv_ref[...], preferred_element_type=jnp.float32)


def _proj(feat, Wq, Wkv):
    rb = 1000
    grid = (N // rb,)
    return pl.pallas_call(
        _proj_body,
        grid=grid,
        in_specs=[
            pl.BlockSpec((rb, F), lambda i: (i, 0)),
            pl.BlockSpec((F, F), lambda i: (0, 0)),
            pl.BlockSpec((F, 2 * F), lambda i: (0, 0)),
        ],
        out_specs=[
            pl.BlockSpec((rb, F), lambda i: (i, 0)),
            pl.BlockSpec((rb, 2 * F), lambda i: (i, 0)),
        ],
        out_shape=[
            jax.ShapeDtypeStruct((N, F), jnp.float32),
            jax.ShapeDtypeStruct((N, 2 * F), jnp.float32),
        ],
    )(feat, Wq, Wkv)


def _sc_body(q_hbm, kv_hbm, src_hbm, dst_hbm, out_hbm,
             srcc, dstc, kv_buf, q_buf, m_buf, zbuf, ft2s, sem):
    cid = lax.axis_index("c")
    sid = lax.axis_index("s")
    wid = cid * NS + sid
    ebase = wid * EW
    iota = lax.iota(jnp.int32, 16)
    zeros16 = jnp.zeros((16,), jnp.float32)

    def zrow(r, carry):
        for c in range(ROWW // 16):
            zbuf[r, pl.ds(c * 16, 16)] = zeros16
        return carry
    lax.fori_loop(0, ZROWS, zrow, 0)
    rbase = sid * RPT
    for i in range(RPT // ZROWS):
        pltpu.sync_copy(zbuf, ft2s.at[pl.ds(rbase + i * ZROWS, ZROWS)])
    plsc.subcore_barrier()

    def chunk_body(ci, carry):
        base = ebase + ci * CHUNK
        pltpu.sync_copy(src_hbm.at[pl.ds(base, CHUNK)], srcc)
        pltpu.sync_copy(dst_hbm.at[pl.ds(base, CHUNK)], dstc)
        cp1 = pltpu.async_copy(kv_hbm.at[srcc], kv_buf, sem)
        cp2 = pltpu.async_copy(q_hbm.at[dstc], q_buf, sem)
        cp1.wait()
        cp2.wait()

        def group_body(g, gcarry):
            rows = g * 16 + iota
            acc = [zeros16, zeros16, zeros16, zeros16]
            for j in range(F):
                h = j // D
                col = jnp.full((16,), j, jnp.int32)
                kvv = plsc.load_gather(kv_buf, [rows, col])
                qv = plsc.load_gather(q_buf, [rows, col])
                acc[h] = acc[h] + kvv * qv
            ee = [jnp.exp(a * INV_SQRT_HD) for a in acc]
            for j in range(F):
                h = j // D
                vcol = jnp.full((16,), F + j, jnp.int32)
                mcol = jnp.full((16,), j, jnp.int32)
                vv = plsc.load_gather(kv_buf, [rows, vcol])
                plsc.store_scatter(m_buf, [rows, mcol], vv * ee[h])
            for h in range(H):
                ecol = jnp.full((16,), F + h, jnp.int32)
                plsc.store_scatter(m_buf, [rows, ecol], ee[h])
            return gcarry
        lax.fori_loop(0, GROUPS, group_body, 0)
        pltpu.sync_copy(m_buf, ft2s.at[dstc], add=True)
        return carry
    lax.fori_loop(0, NCHUNK, chunk_body, 0)

    plsc.subcore_barrier()
    pltpu.sync_copy(ft2s.at[pl.ds(rbase, RPT)],
                    out_hbm.at[cid, pl.ds(rbase, RPT)])


def _sc_edge(Q, KV, src, dst):
    mesh = plsc.VectorSubcoreMesh(core_axis_name="c", subcore_axis_name="s")
    kern = pl.kernel(
        _sc_body,
        out_type=jax.ShapeDtypeStruct((NC, N, ROWW), jnp.float32),
        mesh=mesh,
        scratch_types=[
            pltpu.VMEM((CHUNK,), jnp.int32),
            pltpu.VMEM((CHUNK,), jnp.int32),
            pltpu.VMEM((CHUNK, 2 * F), jnp.float32),
            pltpu.VMEM((CHUNK, F), jnp.float32),
            pltpu.VMEM((CHUNK, ROWW), jnp.float32),
            pltpu.VMEM((ZROWS, ROWW), jnp.float32),
            pltpu.VMEM_SHARED((N, ROWW), jnp.float32),
            pltpu.SemaphoreType.DMA,
        ],
    )
    return kern(Q, KV, src, dst)


def _final_body(feat_ref, p_ref, t_ref, g_ref, b_ref, w1_ref, b1_ref,
                al_ref, w2_ref, b2_ref, o_ref):
    p = p_ref[0] + p_ref[1]
    ft2u = p[:, :F]
    er = jnp.dot(p, t_ref[...], preferred_element_type=jnp.float32)
    ft2 = jnp.where(er > 0.0, ft2u / jnp.maximum(er, 1e-38), 0.0)
    rst = ft2 + feat_ref[...]
    g = g_ref[...]
    b = b_ref[...]

    def ln(x):
        mu = jnp.mean(x, axis=-1, keepdims=True)
        var = jnp.mean((x - mu) ** 2, axis=-1, keepdims=True)
        return (x - mu) * lax.rsqrt(var + 1e-5) * g + b

    rst = ln(rst)
    h = jnp.dot(rst, w1_ref[...], preferred_element_type=jnp.float32)
    h = h + b1_ref[...]
    h = jnp.where(h >= 0.0, h, al_ref[...] * h)
    h = jnp.dot(h, w2_ref[...], preferred_element_type=jnp.float32)
    h = h + b2_ref[...]
    o_ref[...] = ln(rst + h)


def _final(feat, part, T, ln1_g, ln1_b, W1, b1, alpha, W2, b2):
    rb = 1000
    grid = (N // rb,)
    return pl.pallas_call(
        _final_body,
        grid=grid,
        in_specs=[
            pl.BlockSpec((rb, F), lambda i: (i, 0)),
            pl.BlockSpec((NC, rb, ROWW), lambda i: (0, i, 0)),
            pl.BlockSpec((ROWW, F), lambda i: (0, 0)),
            pl.BlockSpec((1, F), lambda i: (0, 0)),
            pl.BlockSpec((1, F), lambda i: (0, 0)),
            pl.BlockSpec((F, 4 * F), lambda i: (0, 0)),
            pl.BlockSpec((1, 4 * F), lambda i: (0, 0)),
            pl.BlockSpec((1, 4 * F), lambda i: (0, 0)),
            pl.BlockSpec((4 * F, F), lambda i: (0, 0)),
            pl.BlockSpec((1, F), lambda i: (0, 0)),
        ],
        out_specs=pl.BlockSpec((rb, F), lambda i: (i, 0)),
        out_shape=jax.ShapeDtypeStruct((N, F), jnp.float32),
    )(feat, part, T, ln1_g.reshape(1, F), ln1_b.reshape(1, F), W1,
      b1.reshape(1, 4 * F), alpha.reshape(1, 4 * F), W2, b2.reshape(1, F))


@jax.jit
def kernel(feat, edge_index, Wq, Wk, Wv, ln1_g, ln1_b, W1, b1, alpha, W2, b2):
    src = edge_index[0]
    dst = edge_index[1]
    Wkv = jnp.concatenate([Wk, Wv], axis=1)
    Q, KV = _proj(feat, Wq, Wkv)
    part = _sc_edge(Q, KV, src, dst)
    T = jnp.zeros((ROWW, F), jnp.float32)
    hsel = jnp.repeat(jnp.arange(H), D)
    T = T.at[F + hsel, jnp.arange(F)].set(1.0)
    return _final(feat, part, T, ln1_g, ln1_b, W1, b1, alpha, W2, b2)


# speedup vs baseline: 13.9102x; 13.9102x over previous
import math

import jax
import jax.numpy as jnp
from jax import lax
from jax.experimental import pallas as pl
from jax.experimental.pallas import tpu as pltpu
from jax.experimental.pallas import tpu_sc as plsc

N = 10000
E = 320000
F = 128
H = 4
D = 32

NC = 2
NS = 16
NW = NC * NS
EW = E // NW
CHUNK = 80
NCHUNK = EW // CHUNK
GROUPS = CHUNK // 16
ROWW = 144
NPAD = 10240
RPT = NPAD // NS
ZROWS = 128
INV_SQRT_HD = 1.0 / math.sqrt(H * D)


def _proj_body(x_ref, wq_ref, wk_ref, wv_ref, q_ref, k_ref, v_ref):
    x = x_ref[...]
    q_ref[...] = jnp.dot(x, wq_ref[...], preferred_element_type=jnp.float32)
    k_ref[...] = jnp.dot(x, wk_ref[...], preferred_element_type=jnp.float32)
    v_ref[...] = jnp.dot(x, wv_ref[...], preferred_element_type=jnp.float32)


def _proj(feat, Wq, Wk, Wv):
    rb = 1000
    grid = (N // rb,)
    w_spec = pl.BlockSpec((F, F), lambda i: (0, 0))
    x_spec = pl.BlockSpec((rb, F), lambda i: (i, 0))
    return pl.pallas_call(
        _proj_body,
        grid=grid,
        in_specs=[x_spec, w_spec, w_spec, w_spec],
        out_specs=[x_spec, x_spec, x_spec],
        out_shape=[jax.ShapeDtypeStruct((N, F), jnp.float32)] * 3,
    )(feat, Wq, Wk, Wv)


def _sc_body(q_hbm, k_hbm, v_hbm, src_hbm, dst_hbm, out_hbm,
             srcc, dstc, k_buf, q_buf, m_buf, ft2s, sem):
    cid = lax.axis_index("c")
    sid = lax.axis_index("s")
    wid = cid * NS + sid
    ebase = wid * EW
    iota = lax.iota(jnp.int32, 16)
    zeros16 = jnp.zeros((16,), jnp.float32)

    def zrow(r, carry):
        for c in range(ROWW // 16):
            m_buf[r, pl.ds(c * 16, 16)] = zeros16
        return carry
    lax.fori_loop(0, CHUNK, zrow, 0)
    rbase = sid * RPT
    for i in range(RPT // CHUNK):
        pltpu.sync_copy(m_buf, ft2s.at[pl.ds(rbase + i * CHUNK, CHUNK)])
    plsc.subcore_barrier()

    @pl.loop(0, NCHUNK)
    def chunk_body(ci):
        base = ebase + ci * CHUNK
        pltpu.sync_copy(src_hbm.at[pl.ds(base, CHUNK)], srcc)
        pltpu.sync_copy(dst_hbm.at[pl.ds(base, CHUNK)], dstc)
        pltpu.async_copy(k_hbm.at[srcc], k_buf, sem).wait()
        pltpu.async_copy(q_hbm.at[dstc], q_buf, sem).wait()

        def group_a(g, gcarry):
            rows = g * 16 + iota
            acc = [zeros16, zeros16, zeros16, zeros16]
            for j in range(F):
                h = j // D
                col = jnp.full((16,), j, jnp.int32)
                kv = plsc.load_gather(k_buf, [rows, col])
                qv = plsc.load_gather(q_buf, [rows, col])
                acc[h] = acc[h] + kv * qv
            for h in range(H):
                ee = jnp.exp(acc[h] * INV_SQRT_HD)
                ecol = jnp.full((16,), F + h, jnp.int32)
                plsc.store_scatter(m_buf, [rows, ecol], ee)
            return gcarry
        lax.fori_loop(0, GROUPS, group_a, 0)

        pltpu.async_copy(v_hbm.at[srcc], q_buf, sem).wait()

        def group_b(g, gcarry):
            rows = g * 16 + iota
            ee = []
            for h in range(H):
                ecol = jnp.full((16,), F + h, jnp.int32)
                ee.append(plsc.load_gather(m_buf, [rows, ecol]))
            for j in range(F):
                h = j // D
                col = jnp.full((16,), j, jnp.int32)
                vv = plsc.load_gather(q_buf, [rows, col])
                plsc.store_scatter(m_buf, [rows, col], vv * ee[h])
            return gcarry
        lax.fori_loop(0, GROUPS, group_b, 0)
        pltpu.sync_copy(m_buf, ft2s.at[dstc], add=True)

    plsc.subcore_barrier()
    pltpu.sync_copy(ft2s.at[pl.ds(rbase, RPT)],
                    out_hbm.at[pl.ds(cid * NPAD + rbase, RPT)])


def _sc_edge(Q, K, V, src, dst):
    mesh = plsc.VectorSubcoreMesh(core_axis_name="c", subcore_axis_name="s")
    kern = pl.kernel(
        _sc_body,
        out_type=jax.ShapeDtypeStruct((NC * NPAD, ROWW), jnp.float32),
        mesh=mesh,
        scratch_types=[
            pltpu.VMEM((CHUNK,), jnp.int32),
            pltpu.VMEM((CHUNK,), jnp.int32),
            pltpu.VMEM((CHUNK, F), jnp.float32),
            pltpu.VMEM((CHUNK, F), jnp.float32),
            pltpu.VMEM((CHUNK, ROWW), jnp.float32),
            pltpu.VMEM_SHARED((NPAD, ROWW), jnp.float32),
            pltpu.SemaphoreType.DMA,
        ],
        compiler_params=pltpu.CompilerParams(
            use_tc_tiling_on_sc=False, needs_layout_passes=False),
    )
    return kern(Q, K, V, src, dst).reshape(NC, NPAD, ROWW)


def _final_body(feat_ref, p_ref, t_ref, g_ref, b_ref, w1_ref, b1_ref,
                al_ref, w2_ref, b2_ref, o_ref):
    p = p_ref[0] + p_ref[1]
    ft2u = p[:, :F]
    er = jnp.dot(p, t_ref[...], preferred_element_type=jnp.float32)
    ft2 = jnp.where(er > 0.0, ft2u / jnp.maximum(er, 1e-38), 0.0)
    rst = ft2 + feat_ref[...]
    g = g_ref[...]
    b = b_ref[...]

    def ln(x):
        mu = jnp.mean(x, axis=-1, keepdims=True)
        var = jnp.mean((x - mu) ** 2, axis=-1, keepdims=True)
        return (x - mu) * lax.rsqrt(var + 1e-5) * g + b

    rst = ln(rst)
    h = jnp.dot(rst, w1_ref[...], preferred_element_type=jnp.float32)
    h = h + b1_ref[...]
    h = jnp.where(h >= 0.0, h, al_ref[...] * h)
    h = jnp.dot(h, w2_ref[...], preferred_element_type=jnp.float32)
    h = h + b2_ref[...]
    o_ref[...] = ln(rst + h)


def _final(feat, part, T, ln1_g, ln1_b, W1, b1, alpha, W2, b2):
    rb = 1000
    grid = (N // rb,)
    return pl.pallas_call(
        _final_body,
        grid=grid,
        in_specs=[
            pl.BlockSpec((rb, F), lambda i: (i, 0)),
            pl.BlockSpec((NC, rb, ROWW), lambda i: (0, i, 0)),
            pl.BlockSpec((ROWW, F), lambda i: (0, 0)),
            pl.BlockSpec((1, F), lambda i: (0, 0)),
            pl.BlockSpec((1, F), lambda i: (0, 0)),
            pl.BlockSpec((F, 4 * F), lambda i: (0, 0)),
            pl.BlockSpec((1, 4 * F), lambda i: (0, 0)),
            pl.BlockSpec((1, 4 * F), lambda i: (0, 0)),
            pl.BlockSpec((4 * F, F), lambda i: (0, 0)),
            pl.BlockSpec((1, F), lambda i: (0, 0)),
        ],
        out_specs=pl.BlockSpec((rb, F), lambda i: (i, 0)),
        out_shape=jax.ShapeDtypeStruct((N, F), jnp.float32),
    )(feat, part, T, ln1_g.reshape(1, F), ln1_b.reshape(1, F), W1,
      b1.reshape(1, 4 * F), alpha.reshape(1, 4 * F), W2, b2.reshape(1, F))


@jax.jit
def kernel(feat, edge_index, Wq, Wk, Wv, ln1_g, ln1_b, W1, b1, alpha, W2, b2):
    src = edge_index[0]
    dst = edge_index[1]
    Q, K, V = _proj(feat, Wq, Wk, Wv)
    part = _sc_edge(Q, K, V, src, dst)
    T = jnp.zeros((ROWW, F), jnp.float32)
    hsel = jnp.repeat(jnp.arange(H), D)
    T = T.at[F + hsel, jnp.arange(F)].set(1.0)
    return _final(feat, part, T, ln1_g, ln1_b, W1, b1, alpha, W2, b2)
